# baseline (speedup 1.0000x reference)
import jax
import jax.numpy as jnp
from jax import lax
from jax.experimental import pallas as pl
from jax.experimental.pallas import tpu as pltpu
from jax.experimental.pallas import tpu_sc as plsc

_MAGIC = 8388608.0

_N = 32 * 1024 * 768
_NC, _NS, _L = 2, 16, 16
_NW = _NC * _NS
_PER_W = _N // _NW
_C = 16384
_NB = 2
_N_CHUNKS = _PER_W // _C
_N_PAIRS = _N_CHUNKS // _NB


def _round16(v):
    vi = v.view(jnp.uint32)
    s = vi & jnp.uint32(0x80000000)
    a = (vi & jnp.uint32(0x7FFFFFFF)).view(jnp.float32)
    r = (a + _MAGIC) - _MAGIC
    r = jnp.where(a < _MAGIC, r, a)
    return (r.view(jnp.uint32) | s).view(jnp.float32)


def _sc_body(x_hbm, o_hbm, inb0, inb1, outb0, outb1, si0, si1, so0, so1):
    inbs, outbs = (inb0, inb1), (outb0, outb1)
    sis, sos = (si0, si1), (so0, so1)
    wid = lax.axis_index("s") * _NC + lax.axis_index("c")
    base = wid * _PER_W

    for b in range(_NB):
        pltpu.make_async_copy(
            x_hbm.at[pl.ds(base + b * _C, _C)], inbs[b], sis[b]
        ).start()

    def pair(g2, _):
        for b in range(_NB):
            off = base + (g2 * _NB + b) * _C
            pltpu.make_async_copy(x_hbm.at[pl.ds(off, _C)], inbs[b], sis[b]).wait()

            @pl.when(g2 > 0)
            def _():
                pltpu.make_async_copy(
                    outbs[b], o_hbm.at[pl.ds(off - _NB * _C, _C)], sos[b]
                ).wait()

            inr, outr = inbs[b], outbs[b]

            @plsc.parallel_loop(0, _C, step=_L, unroll=8)
            def _(i):
                outr[pl.ds(i, _L)] = _round16(inr[pl.ds(i, _L)])

            pltpu.make_async_copy(outr, o_hbm.at[pl.ds(off, _C)], sos[b]).start()

            @pl.when(g2 + 1 < _N_PAIRS)
            def _():
                pltpu.make_async_copy(
                    x_hbm.at[pl.ds(off + _NB * _C, _C)], inbs[b], sis[b]
                ).start()

        return 0

    lax.fori_loop(0, _N_PAIRS, pair, 0)

    for b in range(_NB):
        off_last = base + (_N_CHUNKS - _NB + b) * _C
        pltpu.make_async_copy(outbs[b], o_hbm.at[pl.ds(off_last, _C)], sos[b]).wait()


@jax.jit
def _sc_round(xf):
    mesh = plsc.VectorSubcoreMesh(core_axis_name="c", subcore_axis_name="s")
    f = pl.kernel(
        _sc_body,
        out_type=jax.ShapeDtypeStruct((_N,), jnp.float32),
        mesh=mesh,
        compiler_params=pltpu.CompilerParams(use_tc_tiling_on_sc=True),
        scratch_types=[
            pltpu.VMEM((_C,), jnp.float32),
            pltpu.VMEM((_C,), jnp.float32),
            pltpu.VMEM((_C,), jnp.float32),
            pltpu.VMEM((_C,), jnp.float32),
            pltpu.SemaphoreType.DMA,
            pltpu.SemaphoreType.DMA,
            pltpu.SemaphoreType.DMA,
            pltpu.SemaphoreType.DMA,
        ],
    )
    return f(xf)


def kernel(x):
    B, S, D = x.shape
    return _sc_round(x.reshape(-1)).reshape(B, S, D)


# speedup vs baseline: 3.0635x; 3.0635x over previous
import jax
import jax.numpy as jnp
from jax import lax
from jax.experimental import pallas as pl
from jax.experimental.pallas import tpu as pltpu
from jax.experimental.pallas import tpu_sc as plsc

_MAGIC = 8388608.0

_D = 768
_ROWS = 32 * 1024
_NC, _NS, _L = 2, 16, 16
_NW = _NC * _NS
_ROWS_W = _ROWS // _NW
_R = 32
_NB = 2
_N_CHUNKS = _ROWS_W // _R
_N_PAIRS = _N_CHUNKS // _NB


def _round16(v):
    vi = v.view(jnp.uint32)
    s = vi & jnp.uint32(0x80000000)
    a = (vi & jnp.uint32(0x7FFFFFFF)).view(jnp.float32)
    r = (a + _MAGIC) - _MAGIC
    r = jnp.where(a < _MAGIC, r, a)
    return (r.view(jnp.uint32) | s).view(jnp.float32)


def _sc_body(x_hbm, o_hbm, inb0, inb1, outb0, outb1, si0, si1, so0, so1):
    inbs, outbs = (inb0, inb1), (outb0, outb1)
    sis, sos = (si0, si1), (so0, so1)
    wid = lax.axis_index("s") * _NC + lax.axis_index("c")
    base = wid * _ROWS_W

    for b in range(_NB):
        pltpu.make_async_copy(
            x_hbm.at[pl.ds(base + b * _R, _R), :], inbs[b], sis[b]
        ).start()

    def pair(g2, _):
        for b in range(_NB):
            row0 = base + (g2 * _NB + b) * _R
            pltpu.make_async_copy(
                x_hbm.at[pl.ds(row0, _R), :], inbs[b], sis[b]
            ).wait()

            @pl.when(g2 > 0)
            def _():
                pltpu.make_async_copy(
                    outbs[b], o_hbm.at[pl.ds(row0 - _NB * _R, _R), :], sos[b]
                ).wait()

            inr, outr = inbs[b], outbs[b]

            @plsc.parallel_loop(0, _R * _D, step=_L, unroll=8)
            def _(i):
                r = i // _D
                c = i - r * _D
                outr[r, pl.ds(c, _L)] = _round16(inr[r, pl.ds(c, _L)])

            pltpu.make_async_copy(
                outr, o_hbm.at[pl.ds(row0, _R), :], sos[b]
            ).start()

            @pl.when(g2 + 1 < _N_PAIRS)
            def _():
                pltpu.make_async_copy(
                    x_hbm.at[pl.ds(row0 + _NB * _R, _R), :], inbs[b], sis[b]
                ).start()

        return 0

    lax.fori_loop(0, _N_PAIRS, pair, 0)

    for b in range(_NB):
        row_last = base + (_N_CHUNKS - _NB + b) * _R
        pltpu.make_async_copy(
            outbs[b], o_hbm.at[pl.ds(row_last, _R), :], sos[b]
        ).wait()


@jax.jit
def _sc_round(x2d):
    mesh = plsc.VectorSubcoreMesh(core_axis_name="c", subcore_axis_name="s")
    f = pl.kernel(
        _sc_body,
        out_type=jax.ShapeDtypeStruct((_ROWS, _D), jnp.float32),
        mesh=mesh,
        compiler_params=pltpu.CompilerParams(use_tc_tiling_on_sc=True, skip_device_barrier=True),
        scratch_types=[
            pltpu.VMEM((_R, _D), jnp.float32),
            pltpu.VMEM((_R, _D), jnp.float32),
            pltpu.VMEM((_R, _D), jnp.float32),
            pltpu.VMEM((_R, _D), jnp.float32),
            pltpu.SemaphoreType.DMA,
            pltpu.SemaphoreType.DMA,
            pltpu.SemaphoreType.DMA,
            pltpu.SemaphoreType.DMA,
        ],
    )
    return f(x2d)


def kernel(x):
    B, S, D = x.shape
    return _sc_round(x.reshape(B * S, D)).reshape(B, S, D)


# speedup vs baseline: 3.1266x; 1.0206x over previous
import jax
import jax.numpy as jnp
from jax import lax
from jax.experimental import pallas as pl
from jax.experimental.pallas import tpu as pltpu
from jax.experimental.pallas import tpu_sc as plsc

_MAGIC = 8388608.0

_D = 768
_ROWS = 32 * 1024
_NC, _NS, _L = 2, 16, 16
_NW = _NC * _NS
_ROWS_W = _ROWS // _NW
_R = 16
_NB = 4
_N_CHUNKS = _ROWS_W // _R
_N_GROUPS = _N_CHUNKS // _NB


def _round16(v):
    vi = v.view(jnp.uint32)
    s = vi & jnp.uint32(0x80000000)
    a = (vi & jnp.uint32(0x7FFFFFFF)).view(jnp.float32)
    r = (a + _MAGIC) - _MAGIC
    r = jnp.where(a < _MAGIC, r, a)
    return (r.view(jnp.uint32) | s).view(jnp.float32)


def _sc_body(x_hbm, o_hbm,
             i0, i1, i2, i3, o0, o1, o2, o3,
             si0, si1, si2, si3, so0, so1, so2, so3):
    inbs, outbs = (i0, i1, i2, i3), (o0, o1, o2, o3)
    sis, sos = (si0, si1, si2, si3), (so0, so1, so2, so3)
    wid = lax.axis_index("s") * _NC + lax.axis_index("c")
    base = wid * _ROWS_W

    for b in range(_NB):
        pltpu.make_async_copy(
            x_hbm.at[pl.ds(base + b * _R, _R), :], inbs[b], sis[b]
        ).start()

    def group(g, _):
        for b in range(_NB):
            row0 = base + (g * _NB + b) * _R
            pltpu.make_async_copy(
                x_hbm.at[pl.ds(row0, _R), :], inbs[b], sis[b]
            ).wait()

            @pl.when(g > 0)
            def _():
                pltpu.make_async_copy(
                    outbs[b], o_hbm.at[pl.ds(row0 - _NB * _R, _R), :], sos[b]
                ).wait()

            inr, outr = inbs[b], outbs[b]

            @plsc.parallel_loop(0, _R * _D, step=_L, unroll=8)
            def _(i):
                r = i // _D
                c = i - r * _D
                outr[r, pl.ds(c, _L)] = _round16(inr[r, pl.ds(c, _L)])

            pltpu.make_async_copy(
                outr, o_hbm.at[pl.ds(row0, _R), :], sos[b]
            ).start()

            @pl.when(g + 1 < _N_GROUPS)
            def _():
                pltpu.make_async_copy(
                    x_hbm.at[pl.ds(row0 + _NB * _R, _R), :], inbs[b], sis[b]
                ).start()

        return 0

    lax.fori_loop(0, _N_GROUPS, group, 0)

    for b in range(_NB):
        row_last = base + (_N_CHUNKS - _NB + b) * _R
        pltpu.make_async_copy(
            outbs[b], o_hbm.at[pl.ds(row_last, _R), :], sos[b]
        ).wait()


@jax.jit
def _sc_round(x2d):
    mesh = plsc.VectorSubcoreMesh(core_axis_name="c", subcore_axis_name="s")
    vm = pltpu.VMEM((_R, _D), jnp.float32)
    f = pl.kernel(
        _sc_body,
        out_type=jax.ShapeDtypeStruct((_ROWS, _D), jnp.float32),
        mesh=mesh,
        compiler_params=pltpu.CompilerParams(
            use_tc_tiling_on_sc=True, skip_device_barrier=True
        ),
        scratch_types=[vm] * 8 + [pltpu.SemaphoreType.DMA] * 8,
    )
    return f(x2d)


def kernel(x):
    B, S, D = x.shape
    return _sc_round(x.reshape(B * S, D)).reshape(B, S, D)
